# Initial kernel scaffold; baseline (speedup 1.0000x reference)
import jax
import jax.numpy as jnp
from jax.experimental import pallas as pl


def kernel(x, W_enc, b_enc, dec_bias, topk):
    raise NotImplementedError("write your pallas kernel here")


# speedup vs baseline: 1.5593x; 1.5593x over previous
import functools

import jax
import jax.numpy as jnp
from jax import lax
from jax.experimental import pallas as pl
from jax.experimental.pallas import tpu as pltpu
from jax.experimental.pallas import tpu_sc as plsc

B = 32
D_IN = 4096
WIDTH = 32768
K = 64

WCHUNK = 512
NSTEPS = WIDTH // WCHUNK

GROWS = 8
LANES = 16
VBUF = K + LANES


def _encode_body(x_ref, w_ref, b_ref, z_ref):
    z = lax.dot_general(
        x_ref[...], w_ref[...], (((1,), (1,)), ((), ())),
        preferred_element_type=jnp.float32)
    z_ref[...] = jnp.maximum(z + b_ref[...], 0.0)


def _mask_body(z_ref, masked_ref, i_ref, v_ref):
    z = z_ref[...]
    zb = lax.bitcast_convert_type(z, jnp.int32)
    t = jnp.zeros((B, 1), jnp.int32)
    for bit in range(30, -1, -1):
        cand = t | (1 << bit)
        cnt = jnp.sum((zb >= cand).astype(jnp.int32), axis=1, keepdims=True)
        t = jnp.where(cnt >= K, cand, t)
    m = (zb >= t)
    masked = jnp.where(m, z, 0.0)
    masked_ref[...] = masked

    mcum = m.astype(jnp.int32)
    k = 1
    while k < WIDTH:
        shifted = jnp.concatenate(
            [jnp.zeros((B, k), jnp.int32), mcum[:, :WIDTH - k]], axis=1)
        mcum = mcum + shifted
        k *= 2
    hs, gs = [], []
    for j in range(K + 1):
        s = jnp.clip(mcum - j, 0, 1)
        hs.append(jnp.sum(s, axis=1, keepdims=True))
        gs.append(jnp.sum(masked * s.astype(jnp.float32), axis=1,
                          keepdims=True))
    idx = jnp.concatenate(
        [jnp.minimum(WIDTH - hs[j], WIDTH - 1) for j in range(K)], axis=1)
    val = jnp.concatenate(
        [gs[j] - gs[j + 1] for j in range(K)], axis=1)
    i_ref[...] = idx
    v_ref[...] = val


def _decode_body(idx_hbm, val_hbm, w_hbm, bias_hbm, xhat_hbm,
                 idx_v, val_v, rows_v, acc_v, sem):
    nc = lax.axis_size("c")
    wid = lax.axis_index("s") * nc + lax.axis_index("c")

    pltpu.sync_copy(idx_hbm.at[pl.ds(wid * K, K)], idx_v)
    pltpu.sync_copy(val_hbm.at[pl.ds(wid * K, K)], val_v.at[pl.ds(0, K)])
    pltpu.sync_copy(bias_hbm, acc_v)

    for c in range(K // GROWS):
        pltpu.async_copy(
            w_hbm.at[idx_v.at[pl.ds(c * GROWS, GROWS)]], rows_v, sem).wait()
        vchunk = val_v[pl.ds(c * GROWS, LANES)]
        vals = [jnp.full((LANES,), vchunk[r], jnp.float32)
                for r in range(GROWS)]

        def acc_step(d, _, vals=vals):
            a = acc_v[pl.ds(d * LANES, LANES)]
            for r in range(GROWS):
                a = a + vals[r] * rows_v[r, pl.ds(d * LANES, LANES)]
            acc_v[pl.ds(d * LANES, LANES)] = a
            return 0

        lax.fori_loop(0, D_IN // LANES, acc_step, 0)

    pltpu.sync_copy(acc_v, xhat_hbm.at[pl.ds(wid * D_IN, D_IN)])


@functools.cache
def _make_decode():
    mesh = plsc.VectorSubcoreMesh(
        core_axis_name="c", subcore_axis_name="s",
        num_cores=2, num_subcores=16)
    return pl.kernel(
        _decode_body,
        out_type=jax.ShapeDtypeStruct((B * D_IN,), jnp.float32),
        mesh=mesh,
        scratch_types=[
            pltpu.VMEM((K,), jnp.int32),
            pltpu.VMEM((VBUF,), jnp.float32),
            pltpu.VMEM((GROWS, D_IN), jnp.float32),
            pltpu.VMEM((D_IN,), jnp.float32),
            pltpu.SemaphoreType.DMA,
        ],
    )


def kernel(x, W_enc, b_enc, dec_bias, topk):
    del topk
    z = pl.pallas_call(
        _encode_body,
        grid=(NSTEPS,),
        in_specs=[
            pl.BlockSpec((B, D_IN), lambda i: (0, 0)),
            pl.BlockSpec((WCHUNK, D_IN), lambda i: (i, 0)),
            pl.BlockSpec((1, WCHUNK), lambda i: (0, i)),
        ],
        out_specs=pl.BlockSpec((B, WCHUNK), lambda i: (0, i)),
        out_shape=jax.ShapeDtypeStruct((B, WIDTH), jnp.float32),
    )(x, W_enc, b_enc.reshape(1, WIDTH))

    masked, idcs, vals = pl.pallas_call(
        _mask_body,
        out_shape=(jax.ShapeDtypeStruct((B, WIDTH), jnp.float32),
                   jax.ShapeDtypeStruct((B, K), jnp.int32),
                   jax.ShapeDtypeStruct((B, K), jnp.float32)),
    )(z)

    x_hat = _make_decode()(
        idcs.reshape(B * K), vals.reshape(B * K), W_enc, dec_bias)
    return masked, x_hat.reshape(B, D_IN)
